# Initial kernel scaffold; baseline (speedup 1.0000x reference)
import jax
import jax.numpy as jnp
from jax.experimental import pallas as pl


def kernel(x, t0, t1):
    raise NotImplementedError("write your pallas kernel here")


# speedup vs baseline: 420.1622x; 420.1622x over previous
import functools

import jax
import jax.numpy as jnp
from jax import lax
from jax.experimental import pallas as pl
from jax.experimental.pallas import tpu as pltpu
from jax.experimental.pallas import tpu_sc as plsc

_NUM_BITS = 16
_C = 2
_SCALE = 0.01
_BITS_PER_CHUNK = _NUM_BITS // _C
_CHUNK = 2 ** _BITS_PER_CHUNK
_ZP = 1 << (_NUM_BITS - 1)
_MASK = _CHUNK - 1

_LANES = 16
_NW = 32
_CH = 32768

_RND = 12582912.0


def _sc_body(x_hbm, t0_hbm, t1_hbm, out_hbm, t0_v, t1_v, xbuf, obuf, sem):
    n = x_hbm.shape[0]
    per_w = n // _NW
    n_chunks = per_w // _CH

    cid = lax.axis_index("c")
    sid = lax.axis_index("s")
    wid = sid * 2 + cid
    base = wid * per_w

    pltpu.sync_copy(t0_hbm, t0_v)
    pltpu.sync_copy(t1_hbm, t1_v)

    def inner(i, carry):
        off = i * _LANES
        xv = xbuf[pl.ds(off, _LANES)]
        a = xv / jnp.float32(_SCALE)
        r = (a + _RND) - _RND
        qf = jnp.minimum(jnp.maximum(r + jnp.float32(_ZP), 0.0),
                         jnp.float32(2 ** _NUM_BITS - 1))
        q = qf.astype(jnp.int32)
        i0 = jnp.bitwise_and(q, _MASK)
        i1 = jnp.right_shift(q, _BITS_PER_CHUNK)
        y = plsc.load_gather(t0_v, [i0]) + plsc.load_gather(t1_v, [i1])
        obuf[pl.ds(off, _LANES)] = y
        return carry

    def chunk(c, carry):
        start = base + c * _CH
        pltpu.async_copy(x_hbm.at[pl.ds(start, _CH)], xbuf, sem).wait()
        lax.fori_loop(0, _CH // _LANES, inner, 0, unroll=4)
        pltpu.async_copy(obuf, out_hbm.at[pl.ds(start, _CH)], sem).wait()
        return carry

    lax.fori_loop(0, n_chunks, chunk, 0)


def kernel(x, t0, t1):
    shape = x.shape
    xf = x.reshape(-1)
    n = xf.size
    assert n % (_NW * _CH) == 0

    mesh = plsc.VectorSubcoreMesh(core_axis_name="c", subcore_axis_name="s")
    f = functools.partial(
        pl.kernel,
        out_type=jax.ShapeDtypeStruct((n,), jnp.float32),
        mesh=mesh,
        compiler_params=pltpu.CompilerParams(needs_layout_passes=False),
        scratch_types=[
            pltpu.VMEM((_CHUNK,), jnp.float32),
            pltpu.VMEM((_CHUNK,), jnp.float32),
            pltpu.VMEM((_CH,), jnp.float32),
            pltpu.VMEM((_CH,), jnp.float32),
            pltpu.SemaphoreType.DMA,
        ],
    )(_sc_body)
    out = f(xf, t0, t1)
    return out.reshape(shape)


# speedup vs baseline: 1649.3488x; 3.9255x over previous
import functools

import jax
import jax.numpy as jnp
from jax import lax
from jax.experimental import pallas as pl
from jax.experimental.pallas import tpu as pltpu
from jax.experimental.pallas import tpu_sc as plsc

_NUM_BITS = 16
_SCALE = 0.01
_BITS_PER_CHUNK = 8
_CHUNK = 256
_ZP = 1 << (_NUM_BITS - 1)
_MASK = _CHUNK - 1

_LANES = 16
_NW = 32
_CH = 16384
_UNROLL = 8

_RND = 12582912.0


def _sc_body(x_hbm, t0_hbm, t1_hbm, out_hbm,
             t0_v, t1_v, xb0, xb1, ob0, ob1, si0, si1, so0, so1):
    n = x_hbm.shape[0]
    per_w = n // _NW
    n_chunks = per_w // _CH

    wid = lax.axis_index("s") * 2 + lax.axis_index("c")
    base = wid * per_w

    pltpu.sync_copy(t0_hbm, t0_v)
    pltpu.sync_copy(t1_hbm, t1_v)

    xbs, obs, sis, sos = (xb0, xb1), (ob0, ob1), (si0, si1), (so0, so1)

    def in_copy(c, b):
        return pltpu.make_async_copy(
            x_hbm.at[pl.ds(base + c * _CH, _CH)], xbs[b], sis[b])

    def out_copy(c, b):
        return pltpu.make_async_copy(
            obs[b], out_hbm.at[pl.ds(base + c * _CH, _CH)], sos[b])

    def compute(b):
        xb, ob = xbs[b], obs[b]

        @plsc.parallel_loop(0, _CH, _LANES, unroll=_UNROLL)
        def _(i):
            xv = xb[pl.ds(i, _LANES)]
            a = xv * jnp.float32(1.0 / _SCALE)
            r = (a + jnp.float32(_RND)) - jnp.float32(_RND)
            qf = jnp.minimum(
                jnp.maximum(r + jnp.float32(_ZP), jnp.float32(0.0)),
                jnp.float32(2 ** _NUM_BITS - 1))
            q = qf.astype(jnp.int32)
            i0 = jnp.bitwise_and(q, _MASK)
            i1 = jnp.right_shift(q, _BITS_PER_CHUNK)
            y = plsc.load_gather(t0_v, [i0]) + plsc.load_gather(t1_v, [i1])
            ob[pl.ds(i, _LANES)] = y

    in_copy(0, 0).start()

    def body(c, carry):
        c0 = c * 2
        for b in range(2):
            cc = c0 + b
            nxt = cc + 1

            @pl.when(nxt < n_chunks)
            def _():
                in_copy(nxt, 1 - b).start()

            in_copy(cc, b).wait()

            @pl.when(cc >= 2)
            def _():
                out_copy(cc - 2, b).wait()

            compute(b)
            out_copy(cc, b).start()
        return carry

    lax.fori_loop(0, n_chunks // 2, body, 0)
    out_copy(n_chunks - 2, 0).wait()
    out_copy(n_chunks - 1, 1).wait()


def kernel(x, t0, t1):
    shape = x.shape
    xf = x.reshape(-1)
    n = xf.size
    assert n % (_NW * _CH * 2) == 0

    mesh = plsc.VectorSubcoreMesh(core_axis_name="c", subcore_axis_name="s")
    f = functools.partial(
        pl.kernel,
        out_type=jax.ShapeDtypeStruct((n,), jnp.float32),
        mesh=mesh,
        compiler_params=pltpu.CompilerParams(needs_layout_passes=False),
        scratch_types=[
            pltpu.VMEM((_CHUNK,), jnp.float32),
            pltpu.VMEM((_CHUNK,), jnp.float32),
            pltpu.VMEM((_CH,), jnp.float32),
            pltpu.VMEM((_CH,), jnp.float32),
            pltpu.VMEM((_CH,), jnp.float32),
            pltpu.VMEM((_CH,), jnp.float32),
            pltpu.SemaphoreType.DMA,
            pltpu.SemaphoreType.DMA,
            pltpu.SemaphoreType.DMA,
            pltpu.SemaphoreType.DMA,
        ],
    )(_sc_body)
    out = f(xf, t0, t1)
    return out.reshape(shape)


# speedup vs baseline: 1738.1825x; 1.0539x over previous
import functools

import jax
import jax.numpy as jnp
from jax import lax
from jax.experimental import pallas as pl
from jax.experimental.pallas import tpu as pltpu
from jax.experimental.pallas import tpu_sc as plsc

_NUM_BITS = 16
_SCALE = 0.01
_QMAX = 2 ** _NUM_BITS - 1
_ZP = 1 << (_NUM_BITS - 1)

_LANES = 16
_NW = 32
_CH = 8192
_UNROLL = 8

_RND = 12582912.0


def _sc_body(x_hbm, t01_hbm, out_hbm,
             t01_v, xb0, xb1, ob0, ob1, si0, si1, so0, so1):
    n = x_hbm.shape[0]
    per_w = n // _NW
    n_chunks = per_w // _CH

    wid = lax.axis_index("s") * 2 + lax.axis_index("c")
    base = wid * per_w

    pltpu.sync_copy(t01_hbm, t01_v)

    xbs, obs, sis, sos = (xb0, xb1), (ob0, ob1), (si0, si1), (so0, so1)

    def in_copy(c, b):
        return pltpu.make_async_copy(
            x_hbm.at[pl.ds(base + c * _CH, _CH)], xbs[b], sis[b])

    def out_copy(c, b):
        return pltpu.make_async_copy(
            obs[b], out_hbm.at[pl.ds(base + c * _CH, _CH)], sos[b])

    def compute(b):
        xb, ob = xbs[b], obs[b]

        @plsc.parallel_loop(0, _CH, _LANES, unroll=_UNROLL)
        def _(i):
            xv = xb[pl.ds(i, _LANES)]
            a = xv * jnp.float32(1.0 / _SCALE)
            r = (a + jnp.float32(_RND)) - jnp.float32(_RND)
            qf = jnp.minimum(
                jnp.maximum(r + jnp.float32(_ZP), jnp.float32(0.0)),
                jnp.float32(_QMAX))
            q = qf.astype(jnp.int32)
            ob[pl.ds(i, _LANES)] = plsc.load_gather(t01_v, [q])

    in_copy(0, 0).start()

    def body(c, carry):
        c0 = c * 2
        for b in range(2):
            cc = c0 + b
            nxt = cc + 1

            @pl.when(nxt < n_chunks)
            def _():
                in_copy(nxt, 1 - b).start()

            in_copy(cc, b).wait()

            @pl.when(cc >= 2)
            def _():
                out_copy(cc - 2, b).wait()

            compute(b)
            out_copy(cc, b).start()
        return carry

    lax.fori_loop(0, n_chunks // 2, body, 0)
    out_copy(n_chunks - 2, 0).wait()
    out_copy(n_chunks - 1, 1).wait()


def kernel(x, t0, t1):
    shape = x.shape
    xf = x.reshape(-1)
    n = xf.size
    assert n % (_NW * _CH * 2) == 0

    t01 = (t1[:, None] + t0[None, :]).reshape(-1)

    mesh = plsc.VectorSubcoreMesh(core_axis_name="c", subcore_axis_name="s")
    f = functools.partial(
        pl.kernel,
        out_type=jax.ShapeDtypeStruct((n,), jnp.float32),
        mesh=mesh,
        compiler_params=pltpu.CompilerParams(needs_layout_passes=False),
        scratch_types=[
            pltpu.VMEM((_QMAX + 1,), jnp.float32),
            pltpu.VMEM((_CH,), jnp.float32),
            pltpu.VMEM((_CH,), jnp.float32),
            pltpu.VMEM((_CH,), jnp.float32),
            pltpu.VMEM((_CH,), jnp.float32),
            pltpu.SemaphoreType.DMA,
            pltpu.SemaphoreType.DMA,
            pltpu.SemaphoreType.DMA,
            pltpu.SemaphoreType.DMA,
        ],
    )(_sc_body)
    out = f(xf, t01)
    return out.reshape(shape)


# speedup vs baseline: 3852.1788x; 2.2162x over previous
import functools

import jax
import jax.numpy as jnp
from jax import lax
from jax.experimental import pallas as pl
from jax.experimental.pallas import tpu as pltpu
from jax.experimental.pallas import tpu_sc as plsc

_NUM_BITS = 16
_SCALE = 0.01
_QMAX = 2 ** _NUM_BITS - 1
_ZP = 1 << (_NUM_BITS - 1)

_LANES = 16
_NW = 32
_ROWS = 8
_NBUF = 3

_RND = 12582912.0


def _sc_body(x_hbm, t01_hbm, out_hbm, t01_v, b0, b1, b2,
             si0, si1, si2, so0, so1, so2):
    rows, cols = x_hbm.shape
    rows_w = rows // _NW
    n_chunks = rows_w // _ROWS

    wid = lax.axis_index("s") * 2 + lax.axis_index("c")
    base = wid * rows_w

    pltpu.sync_copy(t01_hbm, t01_v)

    bufs, sis, sos = (b0, b1, b2), (si0, si1, si2), (so0, so1, so2)

    def in_copy(c, b):
        return pltpu.make_async_copy(
            x_hbm.at[pl.ds(base + c * _ROWS, _ROWS), :], bufs[b], sis[b])

    def out_copy(c, b):
        return pltpu.make_async_copy(
            bufs[b], out_hbm.at[pl.ds(base + c * _ROWS, _ROWS), :], sos[b])

    def compute(b):
        buf = bufs[b]
        for r in range(_ROWS):
            @plsc.parallel_loop(0, cols, _LANES, unroll=8)
            def _(i):
                xv = buf[r, pl.ds(i, _LANES)]
                a = xv * jnp.float32(1.0 / _SCALE)
                rr = (a + jnp.float32(_RND)) - jnp.float32(_RND)
                qf = jnp.minimum(
                    jnp.maximum(rr + jnp.float32(_ZP), jnp.float32(0.0)),
                    jnp.float32(_QMAX))
                q = qf.astype(jnp.int32)
                buf[r, pl.ds(i, _LANES)] = plsc.load_gather(t01_v, [q])

    in_copy(0, 0).start()
    in_copy(1, 1).start()

    def step(c, b):
        in_copy(c, b).wait()
        compute(b)
        out_copy(c, b).start()

        @pl.when(c >= 2)
        def _():
            out_copy(c - 2, (b + 1) % _NBUF).wait()

        @pl.when(c + 2 < n_chunks)
        def _():
            in_copy(c + 2, (b + 2) % _NBUF).start()

    def body(g, carry):
        c0 = g * _NBUF
        for b in range(_NBUF):
            step(c0 + b, b)
        return carry

    n_main = n_chunks // _NBUF * _NBUF
    lax.fori_loop(0, n_chunks // _NBUF, body, 0)
    for cc in range(n_main, n_chunks):
        step(cc, cc % _NBUF)

    out_copy(n_chunks - 2, (n_chunks - 2) % _NBUF).wait()
    out_copy(n_chunks - 1, (n_chunks - 1) % _NBUF).wait()


def kernel(x, t0, t1):
    shape = x.shape
    x2 = x.reshape(-1, shape[-1])
    rows, cols = x2.shape
    assert rows % (_NW * _ROWS) == 0 and cols % _LANES == 0

    t01 = (t1[:, None] + t0[None, :]).reshape(-1)

    mesh = plsc.VectorSubcoreMesh(core_axis_name="c", subcore_axis_name="s")
    f = functools.partial(
        pl.kernel,
        out_type=jax.ShapeDtypeStruct((rows, cols), jnp.float32),
        mesh=mesh,
        compiler_params=pltpu.CompilerParams(
            needs_layout_passes=False, use_tc_tiling_on_sc=True),
        scratch_types=[
            pltpu.VMEM((_QMAX + 1,), jnp.float32),
            pltpu.VMEM((_ROWS, cols), jnp.float32),
            pltpu.VMEM((_ROWS, cols), jnp.float32),
            pltpu.VMEM((_ROWS, cols), jnp.float32),
            pltpu.SemaphoreType.DMA,
            pltpu.SemaphoreType.DMA,
            pltpu.SemaphoreType.DMA,
            pltpu.SemaphoreType.DMA,
            pltpu.SemaphoreType.DMA,
            pltpu.SemaphoreType.DMA,
        ],
    )(_sc_body)
    out = f(x2, t01)
    return out.reshape(shape)


# speedup vs baseline: 4822.6739x; 1.2519x over previous
import functools

import jax
import jax.numpy as jnp
from jax import lax
from jax.experimental import pallas as pl
from jax.experimental.pallas import tpu as pltpu
from jax.experimental.pallas import tpu_sc as plsc

_NUM_BITS = 16
_SCALE = 0.01
_QMAX = 2 ** _NUM_BITS - 1
_ZP = 1 << (_NUM_BITS - 1)

_LANES = 16
_NW = 32
_ROWS = 8
_NBUF = 3

_RND = 12582912.0
_RND_BITS = 0x4B400000


def _sc_body(x_hbm, t01_hbm, out_hbm, t01_v, b0, b1, b2,
             si0, si1, si2, so0, so1, so2):
    rows, cols = x_hbm.shape
    rows_w = rows // _NW
    n_chunks = rows_w // _ROWS

    wid = lax.axis_index("s") * 2 + lax.axis_index("c")
    base = wid * rows_w

    pltpu.sync_copy(t01_hbm, t01_v)

    bufs, sis, sos = (b0, b1, b2), (si0, si1, si2), (so0, so1, so2)

    def in_copy(c, b):
        return pltpu.make_async_copy(
            x_hbm.at[pl.ds(base + c * _ROWS, _ROWS), :], bufs[b], sis[b])

    def out_copy(c, b):
        return pltpu.make_async_copy(
            bufs[b], out_hbm.at[pl.ds(base + c * _ROWS, _ROWS), :], sos[b])

    def compute(b):
        buf = bufs[b]
        for r in range(_ROWS):
            @plsc.parallel_loop(0, cols, _LANES, unroll=8)
            def _(i):
                xv = buf[r, pl.ds(i, _LANES)]
                v = xv * jnp.float32(1.0 / _SCALE) + jnp.float32(_RND + _ZP)
                q = plsc.bitcast(v, jnp.int32) - _RND_BITS
                q = jnp.minimum(jnp.maximum(q, 0), _QMAX)
                buf[r, pl.ds(i, _LANES)] = plsc.load_gather(t01_v, [q])

    in_copy(0, 0).start()
    in_copy(1, 1).start()

    def step(c, b):
        in_copy(c, b).wait()
        compute(b)
        out_copy(c, b).start()

        @pl.when(c >= 2)
        def _():
            out_copy(c - 2, (b + 1) % _NBUF).wait()

        @pl.when(c + 2 < n_chunks)
        def _():
            in_copy(c + 2, (b + 2) % _NBUF).start()

    def body(g, carry):
        c0 = g * _NBUF
        for b in range(_NBUF):
            step(c0 + b, b)
        return carry

    n_main = n_chunks // _NBUF * _NBUF
    lax.fori_loop(0, n_chunks // _NBUF, body, 0)
    for cc in range(n_main, n_chunks):
        step(cc, cc % _NBUF)

    out_copy(n_chunks - 2, (n_chunks - 2) % _NBUF).wait()
    out_copy(n_chunks - 1, (n_chunks - 1) % _NBUF).wait()


def kernel(x, t0, t1):
    shape = x.shape
    x2 = x.reshape(-1, shape[-1])
    rows, cols = x2.shape
    assert rows % (_NW * _ROWS) == 0 and cols % _LANES == 0

    t01 = (t1[:, None] + t0[None, :]).reshape(-1)

    mesh = plsc.VectorSubcoreMesh(core_axis_name="c", subcore_axis_name="s")
    f = functools.partial(
        pl.kernel,
        out_type=jax.ShapeDtypeStruct((rows, cols), jnp.float32),
        mesh=mesh,
        compiler_params=pltpu.CompilerParams(
            needs_layout_passes=False, use_tc_tiling_on_sc=True),
        scratch_types=[
            pltpu.VMEM((_QMAX + 1,), jnp.float32),
            pltpu.VMEM((_ROWS, cols), jnp.float32),
            pltpu.VMEM((_ROWS, cols), jnp.float32),
            pltpu.VMEM((_ROWS, cols), jnp.float32),
            pltpu.SemaphoreType.DMA,
            pltpu.SemaphoreType.DMA,
            pltpu.SemaphoreType.DMA,
            pltpu.SemaphoreType.DMA,
            pltpu.SemaphoreType.DMA,
            pltpu.SemaphoreType.DMA,
        ],
    )(_sc_body)
    out = f(x2, t01)
    return out.reshape(shape)
